# baseline (speedup 1.0000x reference)
import functools

import jax
import jax.numpy as jnp
from jax import lax
from jax.experimental import pallas as pl
from jax.experimental.pallas import tpu as pltpu
from jax.experimental.pallas import tpu_sc as plsc

TOPK = 15
ET = 3072
CHUNK = 128
NSEL = 16
NEG_INF = float("-inf")
IMAX = jnp.iinfo(jnp.int32).max


@functools.cache
def _make_sc_gather(v_rows, d, b, idx_per_stream):
    info = plsc.get_sparse_core_info()
    nc, ns = info.num_cores, info.num_subcores
    nw = nc * ns
    assert d % info.num_lanes == 0 and b % (8 * nw) == 0
    b_per_w = b // nw
    assert b_per_w % idx_per_stream == 0
    n_streams = b_per_w // idx_per_stream
    mesh = plsc.VectorSubcoreMesh(core_axis_name="c", subcore_axis_name="s")

    @functools.partial(
        pl.kernel,
        mesh=mesh,
        out_type=jax.ShapeDtypeStruct((b, d), jnp.float32),
        scratch_types=[
            pltpu.VMEM((b_per_w,), jnp.int32),
            pltpu.VMEM((b_per_w, d), jnp.float32),
            pltpu.SemaphoreType.DMA,
        ],
    )
    def gather_kernel(table_hbm, idx_hbm, out_hbm, idx_v, rows_v, sem):
        wid = lax.axis_index("s") * nc + lax.axis_index("c")
        base = wid * b_per_w
        pltpu.sync_copy(idx_hbm.at[pl.ds(base, b_per_w)], idx_v)
        copies = [
            pltpu.async_copy(
                table_hbm.at[idx_v.at[pl.ds(j * idx_per_stream, idx_per_stream)]],
                rows_v.at[pl.ds(j * idx_per_stream, idx_per_stream)],
                sem,
            )
            for j in range(n_streams)
        ]
        for c in copies:
            c.wait()
        pltpu.sync_copy(rows_v, out_hbm.at[pl.ds(base, b_per_w)])

    return gather_kernel


def _simcm_body(n_ent, q, qv_ref, ev_ref, sim_ref, cid_out_ref, simpad_ref,
                cm_ref, copy_sem):
    i = pl.program_id(0)
    nt = pl.num_programs(0)
    nch = ET // CHUNK

    sim = lax.dot_general(qv_ref[...], ev_ref[...],
                          (((1,), (1,)), ((), ())),
                          preferred_element_type=jnp.float32)
    sim_ref[...] = sim
    copies = [
        pltpu.make_async_copy(sim_ref.at[:, c * CHUNK:(c + 1) * CHUNK],
                              simpad_ref.at[c], copy_sem)
        for c in range(nch)
    ]
    for cp in copies:
        cp.start()

    def chunk_max(x):
        return jnp.concatenate(
            [jnp.max(x[:, c * CHUNK:(c + 1) * CHUNK], axis=1, keepdims=True)
             for c in range(nch)], axis=1)

    @pl.when(i < nt - 1)
    def _cm():
        cm_ref[i] = chunk_max(sim)

    @pl.when(i == nt - 1)
    def _cm_last_and_select():
        ids = i * ET + lax.broadcasted_iota(jnp.int32, (q, ET), 1)
        simm = jnp.where(ids < n_ent, sim, NEG_INF)
        cm_ref[i] = chunk_max(simm)

        ncht = nt * nch
        cm = jnp.concatenate([cm_ref[j] for j in range(nt)], axis=1)
        cids = lax.broadcasted_iota(jnp.int32, (q, ncht), 1)
        sels = []
        for _ in range(NSEL):
            m = jnp.max(cm, axis=1, keepdims=True)
            eq = cm == m
            sel = jnp.min(jnp.where(eq, cids, IMAX), axis=1, keepdims=True)
            sels.append(sel)
            cm = jnp.where(eq & (cids == sel), NEG_INF, cm)
        cid_out_ref[...] = jnp.concatenate(sels, axis=1)

    for cp in copies:
        cp.wait()


def _simcm(qv, ev):
    q, d = qv.shape
    n_ent = ev.shape[0]
    nt = (n_ent + ET - 1) // ET
    return pl.pallas_call(
        functools.partial(_simcm_body, n_ent, q),
        grid=(nt,),
        in_specs=[
            pl.BlockSpec((q, d), lambda i: (0, 0)),
            pl.BlockSpec((ET, d), lambda i: (i, 0)),
        ],
        out_specs=[
            pl.BlockSpec((q, ET), lambda i: (0, i)),
            pl.BlockSpec((q, NSEL), lambda i: (0, 0)),
            pl.BlockSpec((ET // CHUNK, q, CHUNK), lambda i: (i, 0, 0)),
        ],
        out_shape=[
            jax.ShapeDtypeStruct((q, n_ent), jnp.float32),
            jax.ShapeDtypeStruct((q, NSEL), jnp.int32),
            jax.ShapeDtypeStruct((nt * (ET // CHUNK), q, CHUNK), jnp.float32),
        ],
        scratch_shapes=[
            pltpu.VMEM((nt, q, ET // CHUNK), jnp.float32),
            pltpu.SemaphoreType.DMA,
        ],
    )(qv, ev)


def _final_body(n_ent, q, cand_ref, cid_ref, topi_ref):
    cand = cand_ref[...]
    cid = cid_ref[...]
    lane = lax.broadcasted_iota(jnp.int32, (q, CHUNK), 1)
    eid = jnp.concatenate(
        [cid[:, c:c + 1] * CHUNK + lane for c in range(NSEL)], axis=1)
    arr = jnp.where(eid < n_ent, cand, NEG_INF)
    idxs = []
    for _ in range(TOPK):
        m = jnp.max(arr, axis=1, keepdims=True)
        eq = arr == m
        sel = jnp.min(jnp.where(eq, eid, IMAX), axis=1, keepdims=True)
        idxs.append(sel)
        arr = jnp.where(eq & (eid == sel), NEG_INF, arr)
    idxs.append(jnp.full((q, 1), IMAX, jnp.int32))
    topi_ref[...] = jnp.concatenate(idxs, axis=1)


def _final_topk(cand, cid, n_ent):
    q = cand.shape[0]
    n_ent_out = 16
    return pl.pallas_call(
        functools.partial(_final_body, n_ent, q),
        grid=(1,),
        in_specs=[
            pl.BlockSpec((q, NSEL * CHUNK), lambda i: (0, 0)),
            pl.BlockSpec((q, NSEL), lambda i: (0, 0)),
        ],
        out_specs=pl.BlockSpec((q, n_ent_out), lambda i: (0, 0)),
        out_shape=jax.ShapeDtypeStruct((q, n_ent_out), jnp.int32),
    )(cand, cid)


def kernel(query_entities, entity_vectors, k):
    n_ent, d = entity_vectors.shape
    q = query_entities.shape[0]

    qv = _make_sc_gather(n_ent, d, q, q // 32)(entity_vectors, query_entities)
    similarity, cid, simpad = _simcm(qv, entity_vectors)

    ncht = simpad.shape[0]
    table = simpad.reshape(ncht * q, CHUNK)
    idx = (cid * q + jnp.arange(q, dtype=jnp.int32)[:, None]).reshape(-1)
    b = q * NSEL
    cand = _make_sc_gather(ncht * q, CHUNK, b, 128)(table, idx)
    cand = cand.reshape(q, NSEL * CHUNK)

    topi16 = _final_topk(cand, cid, n_ent)
    knn_candidates = lax.dynamic_slice_in_dim(topi16, k - 10, TOPK, axis=1)
    return similarity, knn_candidates


# speedup vs baseline: 1.0106x; 1.0106x over previous
import functools

import jax
import jax.numpy as jnp
from jax import lax
from jax.experimental import pallas as pl
from jax.experimental.pallas import tpu as pltpu
from jax.experimental.pallas import tpu_sc as plsc

TOPK = 15
ET = 3072
CHUNK = 128
NSEL = 16
NEG_INF = float("-inf")
IMAX = jnp.iinfo(jnp.int32).max


@functools.cache
def _make_sc_gather(v_rows, d, b, idx_per_stream):
    info = plsc.get_sparse_core_info()
    nc, ns = info.num_cores, info.num_subcores
    nw = nc * ns
    assert d % info.num_lanes == 0 and b % (8 * nw) == 0
    b_per_w = b // nw
    assert b_per_w % idx_per_stream == 0
    n_streams = b_per_w // idx_per_stream
    mesh = plsc.VectorSubcoreMesh(core_axis_name="c", subcore_axis_name="s")

    @functools.partial(
        pl.kernel,
        mesh=mesh,
        out_type=jax.ShapeDtypeStruct((b, d), jnp.float32),
        scratch_types=[
            pltpu.VMEM((b_per_w,), jnp.int32),
            pltpu.VMEM((b_per_w, d), jnp.float32),
            pltpu.SemaphoreType.DMA,
        ],
    )
    def gather_kernel(table_hbm, idx_hbm, out_hbm, idx_v, rows_v, sem):
        wid = lax.axis_index("s") * nc + lax.axis_index("c")
        base = wid * b_per_w
        pltpu.sync_copy(idx_hbm.at[pl.ds(base, b_per_w)], idx_v)
        copies = [
            pltpu.async_copy(
                table_hbm.at[idx_v.at[pl.ds(j * idx_per_stream, idx_per_stream)]],
                rows_v.at[pl.ds(j * idx_per_stream, idx_per_stream)],
                sem,
            )
            for j in range(n_streams)
        ]
        for c in copies:
            c.wait()
        pltpu.sync_copy(rows_v, out_hbm.at[pl.ds(base, b_per_w)])

    return gather_kernel


def _simcm_body(n_ent, q, qv_ref, ev_ref, sim_ref, cid_out_ref, simpad_ref,
                cm_ref):
    i = pl.program_id(0)
    nt = pl.num_programs(0)
    nch = ET // CHUNK

    sim = lax.dot_general(qv_ref[...], ev_ref[...],
                          (((1,), (1,)), ((), ())),
                          preferred_element_type=jnp.float32)
    sim_ref[...] = sim
    for c in range(nch):
        simpad_ref[c] = sim[:, c * CHUNK:(c + 1) * CHUNK]

    def chunk_max(x):
        return jnp.concatenate(
            [jnp.max(x[:, c * CHUNK:(c + 1) * CHUNK], axis=1, keepdims=True)
             for c in range(nch)], axis=1)

    @pl.when(i < nt - 1)
    def _cm():
        cm_ref[i] = chunk_max(sim)

    @pl.when(i == nt - 1)
    def _cm_last_and_select():
        ids = i * ET + lax.broadcasted_iota(jnp.int32, (q, ET), 1)
        simm = jnp.where(ids < n_ent, sim, NEG_INF)
        cm_ref[i] = chunk_max(simm)

        ncht = nt * nch
        cm = jnp.concatenate([cm_ref[j] for j in range(nt)], axis=1)
        cids = lax.broadcasted_iota(jnp.int32, (q, ncht), 1)
        sels = []
        for _ in range(NSEL):
            m = jnp.max(cm, axis=1, keepdims=True)
            eq = cm == m
            sel = jnp.min(jnp.where(eq, cids, IMAX), axis=1, keepdims=True)
            sels.append(sel)
            cm = jnp.where(eq & (cids == sel), NEG_INF, cm)
        cid_out_ref[...] = jnp.concatenate(sels, axis=1)


def _simcm(qv, ev):
    q, d = qv.shape
    n_ent = ev.shape[0]
    nt = (n_ent + ET - 1) // ET
    return pl.pallas_call(
        functools.partial(_simcm_body, n_ent, q),
        grid=(nt,),
        in_specs=[
            pl.BlockSpec((q, d), lambda i: (0, 0)),
            pl.BlockSpec((ET, d), lambda i: (i, 0)),
        ],
        out_specs=[
            pl.BlockSpec((q, ET), lambda i: (0, i)),
            pl.BlockSpec((q, NSEL), lambda i: (0, 0)),
            pl.BlockSpec((ET // CHUNK, q, CHUNK), lambda i: (i, 0, 0)),
        ],
        out_shape=[
            jax.ShapeDtypeStruct((q, n_ent), jnp.float32),
            jax.ShapeDtypeStruct((q, NSEL), jnp.int32),
            jax.ShapeDtypeStruct((nt * (ET // CHUNK), q, CHUNK), jnp.float32),
        ],
        scratch_shapes=[
            pltpu.VMEM((nt, q, ET // CHUNK), jnp.float32),
        ],
    )(qv, ev)


def _final_body(n_ent, q, cand_ref, cid_ref, topi_ref):
    cand = cand_ref[...]
    cid = cid_ref[...]
    lane = lax.broadcasted_iota(jnp.int32, (q, CHUNK), 1)
    eid = jnp.concatenate(
        [cid[:, c:c + 1] * CHUNK + lane for c in range(NSEL)], axis=1)
    arr = jnp.where(eid < n_ent, cand, NEG_INF)
    idxs = []
    for _ in range(TOPK):
        m = jnp.max(arr, axis=1, keepdims=True)
        eq = arr == m
        sel = jnp.min(jnp.where(eq, eid, IMAX), axis=1, keepdims=True)
        idxs.append(sel)
        arr = jnp.where(eq & (eid == sel), NEG_INF, arr)
    idxs.append(jnp.full((q, 1), IMAX, jnp.int32))
    topi_ref[...] = jnp.concatenate(idxs, axis=1)


def _final_topk(cand, cid, n_ent):
    q = cand.shape[0]
    n_ent_out = 16
    return pl.pallas_call(
        functools.partial(_final_body, n_ent, q),
        grid=(1,),
        in_specs=[
            pl.BlockSpec((q, NSEL * CHUNK), lambda i: (0, 0)),
            pl.BlockSpec((q, NSEL), lambda i: (0, 0)),
        ],
        out_specs=pl.BlockSpec((q, n_ent_out), lambda i: (0, 0)),
        out_shape=jax.ShapeDtypeStruct((q, n_ent_out), jnp.int32),
    )(cand, cid)


def kernel(query_entities, entity_vectors, k):
    n_ent, d = entity_vectors.shape
    q = query_entities.shape[0]

    qv = _make_sc_gather(n_ent, d, q, q // 32)(entity_vectors, query_entities)
    similarity, cid, simpad = _simcm(qv, entity_vectors)

    ncht = simpad.shape[0]
    table = simpad.reshape(ncht * q, CHUNK)
    idx = (cid * q + jnp.arange(q, dtype=jnp.int32)[:, None]).reshape(-1)
    b = q * NSEL
    cand = _make_sc_gather(ncht * q, CHUNK, b, 128)(table, idx)
    cand = cand.reshape(q, NSEL * CHUNK)

    topi16 = _final_topk(cand, cid, n_ent)
    knn_candidates = lax.dynamic_slice_in_dim(topi16, k - 10, TOPK, axis=1)
    return similarity, knn_candidates
